# Initial kernel scaffold; baseline (speedup 1.0000x reference)
import jax
import jax.numpy as jnp
from jax.experimental import pallas as pl


def kernel(cls_logits_0, cls_logits_1, cls_logits_2, cls_logits_3, cls_logits_4, cnt_logits_0, cnt_logits_1, cnt_logits_2, cnt_logits_3, cnt_logits_4, reg_preds_0, reg_preds_1, reg_preds_2, reg_preds_3, reg_preds_4, keypoint_preds_0, keypoint_preds_1, keypoint_preds_2, keypoint_preds_3, keypoint_preds_4, gt_boxes, classes, keypoints):
    raise NotImplementedError("write your pallas kernel here")


# speedup vs baseline: 9.3382x; 9.3382x over previous
import functools

import jax
import jax.numpy as jnp
import numpy as np
from jax import lax
from jax.experimental import pallas as pl

_STRIDES = (8, 16, 32, 64, 128)
_LIMITS = ((-1.0, 64.0), (64.0, 128.0), (128.0, 256.0), (256.0, 512.0),
           (512.0, 99999999.0))
_HWS = ((64, 64), (32, 32), (16, 16), (8, 8), (4, 4))
_BIG = 99999999.0

_N = 5456
_NPAD = 5632
_LTILE = 1408
_NT = _NPAD // _LTILE


def _loc_tables():
  xs, ys, rad, lo, hi = [], [], [], [], []
  for (h, w), s, (l0, l1) in zip(_HWS, _STRIDES, _LIMITS):
    sx = np.arange(0, w * s, s, dtype=np.float32) + float(s // 2)
    sy = np.arange(0, h * s, s, dtype=np.float32) + float(s // 2)
    yy, xx = np.meshgrid(sy, sx, indexing='ij')
    xs.append(xx.reshape(-1))
    ys.append(yy.reshape(-1))
    n = h * w
    rad.append(np.full(n, 1.5 * s, np.float32))
    lo.append(np.full(n, l0, np.float32))
    hi.append(np.full(n, l1, np.float32))

  def cat(a):
    v = np.concatenate(a).astype(np.float32)
    v = np.concatenate([v, np.full(_NPAD - v.shape[0], v[-1], np.float32)])
    return v.reshape(_NPAD, 1)

  return cat(xs), cat(ys), cat(rad), cat(lo), cat(hi)


_LOCX, _LOCY, _RAD, _LO, _HI = _loc_tables()


def _body(gtct_ref, gtc_ref, kxy_ref, kvv_ref, locx_ref, locy_ref, rad_ref, lo_ref,
          hi_ref, out_ref):
  gtc = gtc_ref[0]
  kxy = kxy_ref[0]
  kvv = kvv_ref[0]
  x = locx_ref[...]
  y = locy_ref[...]
  rad = rad_ref[...]
  lo = lo_ref[...]
  hi = hi_ref[...]
  L = x.shape[0]

  gtct = gtct_ref[0]
  x0 = gtct[0:1, :]
  y0 = gtct[1:2, :]
  x1 = gtct[2:3, :]
  y1 = gtct[3:4, :]
  gcx = gtct[5:6, :]
  gcy = gtct[6:7, :]

  l = x - x0
  t = y - y0
  r = x1 - x
  b = y1 - y
  areas = (l + r) * (t + b)
  offmin = jnp.minimum(jnp.minimum(l, t), jnp.minimum(r, b))
  offmax = jnp.maximum(jnp.maximum(l, t), jnp.maximum(r, b))
  cmax = jnp.maximum(jnp.abs(x - gcx), jnp.abs(y - gcy))
  mask = ((offmin > 0.0) & (offmax > lo) & (offmax <= hi) & (cmax < rad))
  am = jnp.where(mask, areas, _BIG)
  amin = jnp.min(am, axis=1, keepdims=True)
  ind = jnp.argmin(am, axis=1)
  anym = amin < 1e7

  iot = lax.broadcasted_iota(jnp.int32, (L, 64), 1)
  oh = (iot == ind[:, None]).astype(jnp.float32)

  g1 = jnp.dot(oh, gtc, precision=lax.Precision.HIGHEST,
               preferred_element_type=jnp.float32)
  kpm = jnp.where(kvv == 0.0, -99999.0, kxy)
  g2 = jnp.dot(oh, kpm, precision=lax.Precision.HIGHEST,
               preferred_element_type=jnp.float32)

  lg = x - g1[:, 0:1]
  tg = y - g1[:, 1:2]
  rg = g1[:, 2:3] - x
  bg = g1[:, 3:4] - y
  lrmin = jnp.minimum(lg, rg)
  lrmax = jnp.maximum(lg, rg)
  tbmin = jnp.minimum(tg, bg)
  tbmax = jnp.maximum(tg, bg)
  ratio = lrmin * tbmin / (lrmax * tbmax + 1e-10)
  ratio = jnp.where(anym, ratio, 1.0)
  cnt = jnp.where(anym, jnp.sqrt(ratio), -1.0)
  clso = jnp.where(anym, g1[:, 4:5], 0.0)

  col4 = lax.broadcasted_iota(jnp.int32, (L, 4), 1)
  basexy = jnp.where(col4 % 2 == 0, x, y)
  sgn = jnp.where(col4 < 2, 1.0, -1.0)
  reg = sgn * (basexy - g1[:, 0:4])
  reg = jnp.where(anym, reg, -1.0)

  col34 = lax.broadcasted_iota(jnp.int32, (L, 34), 1)
  basek = jnp.where(col34 % 2 == 0, x, y)
  kq = basek - g2
  kq = jnp.where(kq > 9999.0, -1.0, kq)
  kq = jnp.where(anym, kq, -1.0)

  out_ref[0, :, 0:1] = clso
  out_ref[0, :, 1:2] = cnt
  out_ref[0, :, 2:6] = reg
  out_ref[0, :, 6:40] = kq


@jax.jit
def _run(gt_boxes, classes, keypoints):
  B = gt_boxes.shape[0]
  m = gt_boxes.shape[1]
  gcxy = (gt_boxes[..., 0:2] + gt_boxes[..., 2:4]) * 0.5
  gtc = jnp.concatenate(
      [gt_boxes, classes[..., None].astype(jnp.float32), gcxy,
       jnp.zeros((B, m, 1), jnp.float32)], axis=-1)
  kp3 = keypoints.reshape(B, m, 17, 3)
  kxy = kp3[..., :2].reshape(B, m, 34)
  kvv = jnp.broadcast_to(kp3[..., 2:3], (B, m, 17, 2)).reshape(B, m, 34)

  out = pl.pallas_call(
      _body,
      grid=(B, _NT),
      in_specs=[
          pl.BlockSpec((1, 8, m), lambda bi, ti: (bi, 0, 0)),
          pl.BlockSpec((1, m, 8), lambda bi, ti: (bi, 0, 0)),
          pl.BlockSpec((1, m, 34), lambda bi, ti: (bi, 0, 0)),
          pl.BlockSpec((1, m, 34), lambda bi, ti: (bi, 0, 0)),
          pl.BlockSpec((_LTILE, 1), lambda bi, ti: (ti, 0)),
          pl.BlockSpec((_LTILE, 1), lambda bi, ti: (ti, 0)),
          pl.BlockSpec((_LTILE, 1), lambda bi, ti: (ti, 0)),
          pl.BlockSpec((_LTILE, 1), lambda bi, ti: (ti, 0)),
          pl.BlockSpec((_LTILE, 1), lambda bi, ti: (ti, 0)),
      ],
      out_specs=pl.BlockSpec((1, _LTILE, 40), lambda bi, ti: (bi, ti, 0)),
      out_shape=jax.ShapeDtypeStruct((B, _N, 40), jnp.float32),
  )(jnp.transpose(gtc, (0, 2, 1)), gtc, kxy, kvv, jnp.asarray(_LOCX), jnp.asarray(_LOCY), jnp.asarray(_RAD),
    jnp.asarray(_LO), jnp.asarray(_HI))

  cls_t = out[..., 0:1].astype(jnp.int32)
  cnt_t = out[..., 1:2]
  reg_t = out[..., 2:6]
  kp_t = out[..., 6:40]
  return cls_t, cnt_t, reg_t, kp_t


def kernel(cls_logits_0, cls_logits_1, cls_logits_2, cls_logits_3,
           cls_logits_4, cnt_logits_0, cnt_logits_1, cnt_logits_2,
           cnt_logits_3, cnt_logits_4, reg_preds_0, reg_preds_1, reg_preds_2,
           reg_preds_3, reg_preds_4, keypoint_preds_0, keypoint_preds_1,
           keypoint_preds_2, keypoint_preds_3, keypoint_preds_4, gt_boxes,
           classes, keypoints):
  return _run(gt_boxes, classes, keypoints)
